# Initial kernel scaffold; baseline (speedup 1.0000x reference)
import jax
import jax.numpy as jnp
from jax.experimental import pallas as pl


def kernel(x, label):
    raise NotImplementedError("write your pallas kernel here")


# speedup vs baseline: 11.9177x; 11.9177x over previous
import functools

import jax
import jax.numpy as jnp
from jax import lax
from jax.experimental import pallas as pl

_BIG = 1e9


def _box3(m):
    r = m + jnp.roll(m, 1, axis=1) + jnp.roll(m, -1, axis=1)
    return r + jnp.roll(r, 1, axis=2) + jnp.roll(r, -1, axis=2)


def _ldl_kernel(x_ref, lbl_ref, out_ref, *, B, C, H, W):
    lbl = lbl_ref[...]

    best_v = x_ref[:, 0, :, :]
    best_i = jnp.zeros((B, H, W), jnp.int32)
    for c in range(1, C):
        v = x_ref[:, c, :, :]
        upd = v > best_v
        best_v = jnp.where(upd, v, best_v)
        best_i = jnp.where(upd, c, best_i)
    pred = best_i

    hh = lax.broadcasted_iota(jnp.int32, (B, H, W), 1)
    ww = lax.broadcasted_iota(jnp.int32, (B, H, W), 2)
    interior = (hh >= 1) & (hh <= H - 2) & (ww >= 1) & (ww <= W - 2)

    ma = interior & (_box3(pred) != 9 * pred) & (pred != 0)
    mb = interior & (_box3(lbl) != 9 * lbl)

    ii = lax.broadcasted_iota(jnp.int32, (H, H), 0)
    jj = lax.broadcasted_iota(jnp.int32, (H, H), 1)
    d2 = ((ii - jj) * (ii - jj)).astype(jnp.float32)

    mkey = jnp.where(mb, 0.0, _BIG)
    g = jnp.full((B, H, W), _BIG, jnp.float32)
    for hp in range(H):
        term = d2[:, hp].reshape(1, H, 1) + mkey[:, hp, :].reshape(B, 1, W)
        g = jnp.minimum(g, term)

    md2 = jnp.full((B, H, W), _BIG, jnp.float32)
    for wp in range(W):
        term = g[:, :, wp].reshape(B, H, 1) + d2[wp, :].reshape(1, 1, W)
        md2 = jnp.minimum(md2, term)

    mind = jnp.sqrt(md2)
    anyb = jnp.max(mb.astype(jnp.float32), axis=(1, 2), keepdims=True) > 0.0
    mind = jnp.where(anyb, mind, 0.0)

    maf = ma.astype(jnp.float32)
    na = jnp.sum(maf, axis=(1, 2), keepdims=True)
    s = jnp.sum(maf * mind, axis=(1, 2), keepdims=True)
    loss_b = jnp.where(na > 0.0, s / jnp.maximum(na, 1.0), 0.0)
    loss = jnp.sum(loss_b) / float(B)
    out_ref[...] = jnp.full((1, 128), loss, jnp.float32)


@jax.jit
def kernel(x, label):
    B, C, H, W = x.shape
    out = pl.pallas_call(
        functools.partial(_ldl_kernel, B=B, C=C, H=H, W=W),
        out_shape=jax.ShapeDtypeStruct((1, 128), jnp.float32),
    )(x, label.astype(jnp.int32))
    return out[0, 0]
